# Initial kernel scaffold; baseline (speedup 1.0000x reference)
import jax
import jax.numpy as jnp
from jax.experimental import pallas as pl


def kernel(input):
    raise NotImplementedError("write your pallas kernel here")


# speedup vs baseline: 84.1041x; 84.1041x over previous
import dataclasses
import functools

import jax
import jax.numpy as jnp
from jax import lax
from jax.experimental import pallas as pl
from jax.experimental.pallas import tpu as pltpu
from jax.experimental.pallas import tpu_sc as plsc

N = 2 * 4096 * 4096
K = int(0.9999 * N)
NBINS = 1 << 16
SHIFT = 15

NC, NS = 2, 16
NW = NC * NS

CHUNK = 16384
NBLK = N // CHUNK


def _sc_hist(x2d):
    mesh = plsc.VectorSubcoreMesh(
        core_axis_name="c", subcore_axis_name="s", num_cores=NC, num_subcores=NS
    )

    cp = pltpu.CompilerParams()
    if "needs_layout_passes" in pltpu.CompilerParams.__dataclass_fields__:
        cp = dataclasses.replace(cp, needs_layout_passes=False)

    @functools.partial(
        pl.kernel,
        out_type=jax.ShapeDtypeStruct((NW, NBINS), jnp.int32),
        mesh=mesh,
        scratch_types=[pltpu.VMEM((NBINS,), jnp.int32)],
        compiler_params=cp,
    )
    def hist_kernel(x_hbm, out_hbm, hist):
        zeros16 = jnp.zeros((16,), jnp.int32)

        @pl.loop(0, NBINS, step=16)
        def _zero(i):
            hist[pl.ds(i, 16)] = zeros16

        ones16 = jnp.ones((16,), jnp.int32)

        def body(x_vmem):
            @pl.loop(0, CHUNK, step=16)
            def _(j):
                v = x_vmem[0, pl.ds(j, 16)]
                bits = plsc.bitcast(v, jnp.int32)
                bins = (bits & jnp.int32(0x7FFFFFFF)) >> SHIFT
                plsc.addupdate_scatter(hist, [bins], ones16)

        pltpu.emit_pipeline(
            body,
            grid=(NBLK,),
            in_specs=[pl.BlockSpec((1, CHUNK), lambda i: (i, 0))],
            core_axis_name=("c", "s"),
            dimension_semantics=(pltpu.PARALLEL,),
        )(x_hbm)

        wid = lax.axis_index("s") * NC + lax.axis_index("c")
        pltpu.sync_copy(hist, out_hbm.at[wid])

    return hist_kernel(x2d)


def _tc_select(hists):
    def body(h_ref, o_ref):
        h = h_ref[...].reshape(NW, NBINS // 128, 128)
        m = jnp.sum(h, axis=0)
        r_iota = lax.broadcasted_iota(jnp.int32, m.shape, 0)
        c_iota = lax.broadcasted_iota(jnp.int32, m.shape, 1)
        bin_idx = r_iota * 128 + c_iota

        def bs_body(_, carry):
            lo, hi = carry
            mid = (lo + hi) // 2
            s = jnp.sum(jnp.where(bin_idx <= mid, m, 0))
            ge = s >= K
            return jnp.where(ge, lo, mid + 1), jnp.where(ge, mid, hi)

        lo, _ = lax.fori_loop(
            0, 17, bs_body, (jnp.int32(0), jnp.int32(NBINS - 1))
        )
        lower_bits = lo << SHIFT
        upper_bits = jnp.minimum((lo + 1) << SHIFT, jnp.int32(0x7F7FFFFF))
        lower = lax.bitcast_convert_type(lower_bits, jnp.float32)
        upper = lax.bitcast_convert_type(upper_bits, jnp.float32)
        o_ref[...] = jnp.full((1, 1), 0.5 * (lower + upper), jnp.float32)

    return pl.pallas_call(
        body,
        out_shape=jax.ShapeDtypeStruct((1, 1), jnp.float32),
    )(hists)


def kernel(input):
    x2d = input.reshape(NBLK, CHUNK)
    hists = _sc_hist(x2d)
    out = _tc_select(hists)
    return out.reshape(1)
